# baseline (speedup 1.0000x reference)
import functools

import jax
import jax.numpy as jnp
from jax.experimental import pallas as pl
from jax.experimental.pallas import tpu as pltpu

_ROW_BLOCK = 8


def _lse_body(x_ref, a_ref, out_ref, *, v_total):
    x = x_ref[...]
    col = jax.lax.broadcasted_iota(jnp.int32, x.shape, 1)
    x = jnp.where(col < v_total, x, -jnp.inf)

    a = a_ref[...]
    g = jnp.sum(jnp.where(col == a, x, 0.0), axis=1, keepdims=True)

    m = jnp.max(x, axis=1, keepdims=True)
    s = jnp.sum(jnp.exp(x - m), axis=1, keepdims=True)
    out_ref[...] = g - (m + jnp.log(s))


def kernel(logits, actions):
    b, v = logits.shape
    a = actions.astype(jnp.int32)
    nb = b // _ROW_BLOCK
    return pl.pallas_call(
        functools.partial(_lse_body, v_total=v),
        grid=(nb,),
        in_specs=[
            pl.BlockSpec((_ROW_BLOCK, v), lambda i: (i, 0)),
            pl.BlockSpec((_ROW_BLOCK, 1), lambda i: (i, 0)),
        ],
        out_specs=pl.BlockSpec((_ROW_BLOCK, 1), lambda i: (i, 0)),
        out_shape=jax.ShapeDtypeStruct((b, 1), jnp.float32),
    )(logits, a)
